# Initial kernel scaffold; baseline (speedup 1.0000x reference)
import jax
import jax.numpy as jnp
from jax.experimental import pallas as pl


def kernel(x, edge_index, W1, b1, W2, b2):
    raise NotImplementedError("write your pallas kernel here")


# speedup vs baseline: 183.7902x; 183.7902x over previous
import functools

import jax
import jax.numpy as jnp
from jax import lax
from jax.experimental import pallas as pl
from jax.experimental.pallas import tpu as pltpu
from jax.experimental.pallas import tpu_sc as plsc

N = 100000
E = 6400000
H = 32

NPAD = 102400
NC = 2
NS = 16
NW = NC * NS
ROW = 128
CROWS = 16
CHUNK = CROWS * ROW
NCHT = E // CHUNK
ITERS = -(-NCHT // NW)
SLICE = NPAD // NS
LANES = 16

ROWS2D = NPAD // 128

_mesh = plsc.VectorSubcoreMesh(core_axis_name="c", subcore_axis_name="s")


def _fill1d(ref, n, value):
    v = jnp.full((LANES,), value, jnp.float32)

    def body(i, _):
        ref[pl.ds(i * LANES, LANES)] = v
        return 0

    lax.fori_loop(0, n // LANES, body, 0)


def _fill2d(ref, value):
    v = jnp.full((LANES,), value, jnp.float32)
    for j in range(CROWS):
        def body(l, _):
            ref[j, pl.ds(l * LANES, LANES)] = v
            return 0
        lax.fori_loop(0, ROW // LANES, body, 0)


def _init_acc_slice(zb_v, acc_sh, s):
    _fill1d(zb_v, SLICE, 0.0)
    pltpu.sync_copy(zb_v, acc_sh.at[pl.ds(s * SLICE, SLICE)])


def _writeout(zb_v, acc_sh, out_hbm, c, s):
    pltpu.sync_copy(acc_sh.at[pl.ds(s * SLICE, SLICE)], zb_v)
    pltpu.sync_copy(zb_v, out_hbm.at[c, pl.ds(s * SLICE, SLICE)])


@functools.partial(
    pl.kernel,
    out_type=jax.ShapeDtypeStruct((NC, NPAD), jnp.float32),
    mesh=_mesh,
    scratch_types=[
        pltpu.VMEM((CROWS, ROW), jnp.int32),
        pltpu.VMEM((CROWS, ROW), jnp.float32),
        pltpu.VMEM((SLICE,), jnp.float32),
        pltpu.VMEM_SHARED((NPAD,), jnp.float32),
        pltpu.SemaphoreType.DMA,
    ],
)
def _deg_kernel(dst_hbm, out_hbm, idx_v, val_v, zb_v, acc_sh, sem):
    c = lax.axis_index("c")
    s = lax.axis_index("s")
    wid = c * NS + s

    _init_acc_slice(zb_v, acc_sh, s)
    _fill2d(val_v, 1.0)
    plsc.subcore_barrier()

    def chunk_body(i, _):
        k = wid + i * NW

        @pl.when(k < NCHT)
        def _():
            pltpu.sync_copy(dst_hbm.at[k], idx_v)
            descs = []
            for j in range(CROWS):
                descs.append(
                    pltpu.async_copy(val_v.at[j], acc_sh.at[idx_v.at[j]], sem,
                                     add=True))
            for d in descs:
                d.wait()

        return 0

    lax.fori_loop(0, ITERS, chunk_body, 0)

    plsc.subcore_barrier()
    _writeout(zb_v, acc_sh, out_hbm, c, s)


@functools.partial(
    pl.kernel,
    out_type=jax.ShapeDtypeStruct((NC, NPAD), jnp.float32),
    mesh=_mesh,
    scratch_types=[
        pltpu.VMEM((NPAD,), jnp.float32),
        pltpu.VMEM((CHUNK,), jnp.int32),
        pltpu.VMEM((CROWS, ROW), jnp.int32),
        pltpu.VMEM((CHUNK,), jnp.float32),
        pltpu.VMEM((SLICE,), jnp.float32),
        pltpu.VMEM_SHARED((NPAD,), jnp.float32),
        pltpu.SemaphoreType.DMA,
    ],
    compiler_params=pltpu.CompilerParams(needs_layout_passes=False),
)
def _edge_sum_kernel(srcf_hbm, dst_hbm, g_hbm, out_hbm, g_v, src_v, dst_v,
                     val_v, zb_v, acc_sh, sem):
    c = lax.axis_index("c")
    s = lax.axis_index("s")
    wid = c * NS + s

    pltpu.sync_copy(g_hbm, g_v)
    _init_acc_slice(zb_v, acc_sh, s)
    plsc.subcore_barrier()

    def chunk_body(i, _):
        k = wid + i * NW

        @pl.when(k < NCHT)
        def _():
            pltpu.sync_copy(srcf_hbm.at[k], src_v)
            pltpu.sync_copy(dst_hbm.at[k], dst_v)

            def gather_body(l, _):
                idx = src_v[pl.ds(l * LANES, LANES)]
                val_v[pl.ds(l * LANES, LANES)] = plsc.load_gather(g_v, [idx])
                return 0

            lax.fori_loop(0, CHUNK // LANES, gather_body, 0)
            descs = []
            for j in range(CROWS):
                descs.append(
                    pltpu.async_copy(val_v.at[pl.ds(j * ROW, ROW)],
                                     acc_sh.at[dst_v.at[j]], sem, add=True))
            for d in descs:
                d.wait()

        return 0

    lax.fori_loop(0, ITERS, chunk_body, 0)

    plsc.subcore_barrier()
    _writeout(zb_v, acc_sh, out_hbm, c, s)


def _node_prep_body(deg_parts_ref, x_ref, g_ref, dinv_ref):
    deg = deg_parts_ref[0] + deg_parts_ref[1] + 1.0
    dinv = lax.rsqrt(deg)
    dinv_ref[...] = dinv
    g_ref[...] = x_ref[...] * dinv


_node_prep = pl.pallas_call(
    _node_prep_body,
    out_shape=(
        jax.ShapeDtypeStruct((ROWS2D, 128), jnp.float32),
        jax.ShapeDtypeStruct((ROWS2D, 128), jnp.float32),
    ),
)


def _node_final_body(t_parts_ref, g_ref, dinv_ref, x_ref, w1_ref, b1_ref,
                     w2_ref, b2_ref, out_ref):
    t = t_parts_ref[0] + t_parts_ref[1]
    sc = dinv_ref[...] * (t + g_ref[...])
    acc = jnp.zeros((ROWS2D, 128), jnp.float32)
    for j in range(H):
        h = jnp.maximum(sc * w1_ref[0, j] + b1_ref[0, j], 0.0)
        acc = acc + h * w2_ref[0, j]
    out_ref[...] = x_ref[...] + acc + b2_ref[0, 0]


_node_final = pl.pallas_call(
    _node_final_body,
    in_specs=[
        pl.BlockSpec(memory_space=pltpu.VMEM),
        pl.BlockSpec(memory_space=pltpu.VMEM),
        pl.BlockSpec(memory_space=pltpu.VMEM),
        pl.BlockSpec(memory_space=pltpu.VMEM),
        pl.BlockSpec(memory_space=pltpu.SMEM),
        pl.BlockSpec(memory_space=pltpu.SMEM),
        pl.BlockSpec(memory_space=pltpu.SMEM),
        pl.BlockSpec(memory_space=pltpu.SMEM),
    ],
    out_shape=jax.ShapeDtypeStruct((ROWS2D, 128), jnp.float32),
)


def kernel(x, edge_index, W1, b1, W2, b2):
    srcf = edge_index[0].astype(jnp.int32).reshape(NCHT, CHUNK)
    dst = edge_index[1].astype(jnp.int32).reshape(NCHT, CROWS, ROW)
    x_flat = x[:, 0]
    x_pad = jnp.zeros((NPAD,), jnp.float32).at[:N].set(x_flat)

    deg_parts = _deg_kernel(dst)
    g, dinv = _node_prep(deg_parts.reshape(NC, ROWS2D, 128),
                         x_pad.reshape(ROWS2D, 128))

    t_parts = _edge_sum_kernel(srcf, dst, g.reshape(NPAD))

    out_pad = _node_final(
        t_parts.reshape(NC, ROWS2D, 128),
        g,
        dinv,
        x_pad.reshape(ROWS2D, 128),
        W1.reshape(1, H),
        b1.reshape(1, H),
        W2.reshape(1, H),
        b2.reshape(1, 1),
    )
    return out_pad.reshape(NPAD)[:N].reshape(N, 1)


# speedup vs baseline: 259.9634x; 1.4145x over previous
import functools

import jax
import jax.numpy as jnp
from jax import lax
from jax.experimental import pallas as pl
from jax.experimental.pallas import tpu as pltpu
from jax.experimental.pallas import tpu_sc as plsc

N = 100000
E = 6400000
H = 32

NPAD = 102400
NC = 2
NS = 16
NW = NC * NS
ROW = 128
CROWS = 16
CHUNK = CROWS * ROW
NCHT = E // CHUNK
ITERS = -(-NCHT // NW)
SLICE = NPAD // NS
LANES = 16

ROWS2D = NPAD // 128

_mesh = plsc.VectorSubcoreMesh(core_axis_name="c", subcore_axis_name="s")


def _fill1d(ref, n, value):
    v = jnp.full((LANES,), value, jnp.float32)

    def body(i, _):
        ref[pl.ds(i * LANES, LANES)] = v
        return 0

    lax.fori_loop(0, n // LANES, body, 0)


def _init_acc_slice(zb_v, acc_sh, s):
    _fill1d(zb_v, SLICE, 0.0)
    pltpu.sync_copy(zb_v, acc_sh.at[pl.ds(s * SLICE, SLICE)])


def _writeout(zb_v, acc_sh, out_hbm, c, s):
    pltpu.sync_copy(acc_sh.at[pl.ds(s * SLICE, SLICE)], zb_v)
    pltpu.sync_copy(zb_v, out_hbm.at[c, pl.ds(s * SLICE, SLICE)])


@functools.partial(
    pl.kernel,
    out_type=jax.ShapeDtypeStruct((NC, NPAD), jnp.float32),
    mesh=_mesh,
    scratch_types=[
        pltpu.VMEM((CROWS, ROW), jnp.int32),
        pltpu.VMEM((CROWS, ROW), jnp.int32),
        pltpu.VMEM((CHUNK,), jnp.float32),
        pltpu.VMEM((SLICE,), jnp.float32),
        pltpu.VMEM_SHARED((NPAD,), jnp.float32),
        pltpu.SemaphoreType.DMA,
        pltpu.SemaphoreType.DMA,
        pltpu.SemaphoreType.DMA,
    ],
    compiler_params=pltpu.CompilerParams(needs_layout_passes=False),
)
def _deg_kernel(dst_hbm, out_hbm, dst0_v, dst1_v, ones_v, zb_v, acc_sh, semA,
                semB, sem_s):
    c = lax.axis_index("c")
    s = lax.axis_index("s")
    wid = c * NS + s

    _init_acc_slice(zb_v, acc_sh, s)
    _fill1d(ones_v, CHUNK, 1.0)
    plsc.subcore_barrier()

    def fire_scatters(dst_v):
        return [
            pltpu.async_copy(ones_v.at[pl.ds(j * ROW, ROW)],
                             acc_sh.at[dst_v.at[j]], sem_s, add=True)
            for j in range(CROWS)
        ]

    pltpu.async_copy(dst_hbm.at[wid], dst0_v, semA)

    def pair_body(i, _):
        kA = wid + (2 * i) * NW
        kB = kA + NW
        kA2 = kA + 2 * NW

        @pl.when(kB < NCHT)
        def _():
            pltpu.async_copy(dst_hbm.at[kB], dst1_v, semB)

        pltpu.make_async_copy(dst_hbm.at[kA], dst0_v, semA).wait()
        descsA = fire_scatters(dst0_v)

        @pl.when(kB < NCHT)
        def _():
            pltpu.make_async_copy(dst_hbm.at[kB], dst1_v, semB).wait()
            descsB = fire_scatters(dst1_v)
            for d in descsA:
                d.wait()

            @pl.when(kA2 < NCHT)
            def _():
                pltpu.async_copy(dst_hbm.at[kA2], dst0_v, semA)

            for d in descsB:
                d.wait()

        @pl.when(kB >= NCHT)
        def _():
            for d in descsA:
                d.wait()

        return 0

    lax.fori_loop(0, (ITERS + 1) // 2, pair_body, 0)

    plsc.subcore_barrier()
    _writeout(zb_v, acc_sh, out_hbm, c, s)


@functools.partial(
    pl.kernel,
    out_type=jax.ShapeDtypeStruct((NC, NPAD), jnp.float32),
    mesh=_mesh,
    scratch_types=[
        pltpu.VMEM((NPAD,), jnp.float32),
        pltpu.VMEM((CHUNK,), jnp.int32),
        pltpu.VMEM((CHUNK,), jnp.int32),
        pltpu.VMEM((CROWS, ROW), jnp.int32),
        pltpu.VMEM((CROWS, ROW), jnp.int32),
        pltpu.VMEM((CHUNK,), jnp.float32),
        pltpu.VMEM((CHUNK,), jnp.float32),
        pltpu.VMEM((SLICE,), jnp.float32),
        pltpu.VMEM_SHARED((NPAD,), jnp.float32),
        pltpu.SemaphoreType.DMA,
        pltpu.SemaphoreType.DMA,
        pltpu.SemaphoreType.DMA,
    ],
    compiler_params=pltpu.CompilerParams(needs_layout_passes=False),
)
def _edge_sum_kernel(srcf_hbm, dst_hbm, g_hbm, out_hbm, g_v, src0_v, src1_v,
                     dst0_v, dst1_v, val0_v, val1_v, zb_v, acc_sh, semA, semB,
                     sem_s):
    c = lax.axis_index("c")
    s = lax.axis_index("s")
    wid = c * NS + s

    pltpu.sync_copy(g_hbm, g_v)
    _init_acc_slice(zb_v, acc_sh, s)
    plsc.subcore_barrier()

    def gather_chunk(src_v, val_v):
        def gather_body(l, _):
            for u in range(4):
                o = (l * 4 + u) * LANES
                idx = src_v[pl.ds(o, LANES)]
                val_v[pl.ds(o, LANES)] = plsc.load_gather(g_v, [idx])
            return 0

        lax.fori_loop(0, CHUNK // (4 * LANES), gather_body, 0)

    def fire_scatters(dst_v, val_v):
        return [
            pltpu.async_copy(val_v.at[pl.ds(j * ROW, ROW)],
                             acc_sh.at[dst_v.at[j]], sem_s, add=True)
            for j in range(CROWS)
        ]

    def fire_fetch(k, src_v, dst_v, sem):
        pltpu.async_copy(srcf_hbm.at[k], src_v, sem)
        pltpu.async_copy(dst_hbm.at[k], dst_v, sem)

    def drain_fetch(k, src_v, dst_v, sem):
        pltpu.make_async_copy(srcf_hbm.at[k], src_v, sem).wait()
        pltpu.make_async_copy(dst_hbm.at[k], dst_v, sem).wait()

    fire_fetch(wid, src0_v, dst0_v, semA)

    def pair_body(i, _):
        kA = wid + (2 * i) * NW
        kB = kA + NW
        kA2 = kA + 2 * NW

        @pl.when(kB < NCHT)
        def _():
            fire_fetch(kB, src1_v, dst1_v, semB)

        drain_fetch(kA, src0_v, dst0_v, semA)
        gather_chunk(src0_v, val0_v)
        descsA = fire_scatters(dst0_v, val0_v)

        @pl.when(kB < NCHT)
        def _():
            drain_fetch(kB, src1_v, dst1_v, semB)
            gather_chunk(src1_v, val1_v)
            descsB = fire_scatters(dst1_v, val1_v)
            for d in descsA:
                d.wait()

            @pl.when(kA2 < NCHT)
            def _():
                fire_fetch(kA2, src0_v, dst0_v, semA)

            for d in descsB:
                d.wait()

        @pl.when(kB >= NCHT)
        def _():
            for d in descsA:
                d.wait()

        return 0

    lax.fori_loop(0, (ITERS + 1) // 2, pair_body, 0)

    plsc.subcore_barrier()
    _writeout(zb_v, acc_sh, out_hbm, c, s)


def _node_prep_body(deg_parts_ref, x_ref, g_ref, dinv_ref):
    deg = deg_parts_ref[0] + deg_parts_ref[1] + 1.0
    dinv = lax.rsqrt(deg)
    dinv_ref[...] = dinv
    g_ref[...] = x_ref[...] * dinv


_node_prep = pl.pallas_call(
    _node_prep_body,
    out_shape=(
        jax.ShapeDtypeStruct((ROWS2D, 128), jnp.float32),
        jax.ShapeDtypeStruct((ROWS2D, 128), jnp.float32),
    ),
)


def _node_final_body(t_parts_ref, g_ref, dinv_ref, x_ref, w1_ref, b1_ref,
                     w2_ref, b2_ref, out_ref):
    t = t_parts_ref[0] + t_parts_ref[1]
    sc = dinv_ref[...] * (t + g_ref[...])
    acc = jnp.zeros((ROWS2D, 128), jnp.float32)
    for j in range(H):
        h = jnp.maximum(sc * w1_ref[0, j] + b1_ref[0, j], 0.0)
        acc = acc + h * w2_ref[0, j]
    out_ref[...] = x_ref[...] + acc + b2_ref[0, 0]


_node_final = pl.pallas_call(
    _node_final_body,
    in_specs=[
        pl.BlockSpec(memory_space=pltpu.VMEM),
        pl.BlockSpec(memory_space=pltpu.VMEM),
        pl.BlockSpec(memory_space=pltpu.VMEM),
        pl.BlockSpec(memory_space=pltpu.VMEM),
        pl.BlockSpec(memory_space=pltpu.SMEM),
        pl.BlockSpec(memory_space=pltpu.SMEM),
        pl.BlockSpec(memory_space=pltpu.SMEM),
        pl.BlockSpec(memory_space=pltpu.SMEM),
    ],
    out_shape=jax.ShapeDtypeStruct((ROWS2D, 128), jnp.float32),
)


def kernel(x, edge_index, W1, b1, W2, b2):
    srcf = edge_index[0].astype(jnp.int32).reshape(NCHT, CHUNK)
    dstf = edge_index[1].astype(jnp.int32)
    dst = dstf.reshape(NCHT, CROWS, ROW)
    x_flat = x[:, 0]
    x_pad = jnp.zeros((NPAD,), jnp.float32).at[:N].set(x_flat)

    deg_parts = _deg_kernel(dst)
    g, dinv = _node_prep(deg_parts.reshape(NC, ROWS2D, 128),
                         x_pad.reshape(ROWS2D, 128))

    t_parts = _edge_sum_kernel(srcf, dst, g.reshape(NPAD))

    out_pad = _node_final(
        t_parts.reshape(NC, ROWS2D, 128),
        g,
        dinv,
        x_pad.reshape(ROWS2D, 128),
        W1.reshape(1, H),
        b1.reshape(1, H),
        W2.reshape(1, H),
        b2.reshape(1, 1),
    )
    return out_pad.reshape(NPAD)[:N].reshape(N, 1)


# speedup vs baseline: 491.9629x; 1.8924x over previous
import functools

import jax
import jax.numpy as jnp
from jax import lax
from jax.experimental import pallas as pl
from jax.experimental.pallas import tpu as pltpu
from jax.experimental.pallas import tpu_sc as plsc

N = 100000
E = 6400000
H = 32

NPAD = 102400
NC = 2
NS = 16
NW = NC * NS
ROW = 128
CROWS = 16
CHUNK = CROWS * ROW
NCHT = E // CHUNK
ITERS = -(-NCHT // NW)
SLICE = NPAD // NS
LANES = 16

ROWS2D = NPAD // 128

_mesh = plsc.VectorSubcoreMesh(core_axis_name="c", subcore_axis_name="s")


def _fill1d(ref, n, value):
    v = jnp.full((LANES,), value, jnp.float32)

    def body(i, _):
        ref[pl.ds(i * LANES, LANES)] = v
        return 0

    lax.fori_loop(0, n // LANES, body, 0)


def _init_acc_slice(zb_v, acc_sh, s):
    _fill1d(zb_v, SLICE, 0.0)
    pltpu.sync_copy(zb_v, acc_sh.at[pl.ds(s * SLICE, SLICE)])


def _writeout(zb_v, acc_sh, out_hbm, c, s):
    pltpu.sync_copy(acc_sh.at[pl.ds(s * SLICE, SLICE)], zb_v)
    pltpu.sync_copy(zb_v, out_hbm.at[c, pl.ds(s * SLICE, SLICE)])


@functools.partial(
    pl.kernel,
    out_type=jax.ShapeDtypeStruct((NC, NPAD), jnp.float32),
    mesh=_mesh,
    scratch_types=[
        pltpu.VMEM((CROWS, ROW), jnp.int32),
        pltpu.VMEM((CROWS, ROW), jnp.int32),
        pltpu.VMEM((CHUNK,), jnp.float32),
        pltpu.VMEM((SLICE,), jnp.float32),
        pltpu.VMEM_SHARED((NPAD,), jnp.float32),
        pltpu.SemaphoreType.DMA,
        pltpu.SemaphoreType.DMA,
        pltpu.SemaphoreType.DMA,
    ],
    compiler_params=pltpu.CompilerParams(needs_layout_passes=False),
)
def _deg_kernel(dst_hbm, out_hbm, dst0_v, dst1_v, ones_v, zb_v, acc_sh, semA,
                semB, sem_s):
    c = lax.axis_index("c")
    s = lax.axis_index("s")
    wid = c * NS + s

    _init_acc_slice(zb_v, acc_sh, s)
    _fill1d(ones_v, CHUNK, 1.0)
    plsc.subcore_barrier()

    def fire_scatters(dst_v):
        return [
            pltpu.async_copy(ones_v.at[pl.ds(j * ROW, ROW)],
                             acc_sh.at[dst_v.at[j]], sem_s, add=True)
            for j in range(CROWS)
        ]

    pltpu.async_copy(dst_hbm.at[wid], dst0_v, semA)

    def pair_body(i, _):
        kA = wid + (2 * i) * NW
        kB = kA + NW
        kA2 = kA + 2 * NW

        @pl.when(kB < NCHT)
        def _():
            pltpu.async_copy(dst_hbm.at[kB], dst1_v, semB)

        pltpu.make_async_copy(dst_hbm.at[kA], dst0_v, semA).wait()
        descsA = fire_scatters(dst0_v)

        @pl.when(kB < NCHT)
        def _():
            pltpu.make_async_copy(dst_hbm.at[kB], dst1_v, semB).wait()
            descsB = fire_scatters(dst1_v)
            for d in descsA:
                d.wait()

            @pl.when(kA2 < NCHT)
            def _():
                pltpu.async_copy(dst_hbm.at[kA2], dst0_v, semA)

            for d in descsB:
                d.wait()

        @pl.when(kB >= NCHT)
        def _():
            for d in descsA:
                d.wait()

        return 0

    lax.fori_loop(0, (ITERS + 1) // 2, pair_body, 0)

    plsc.subcore_barrier()
    _writeout(zb_v, acc_sh, out_hbm, c, s)


@functools.partial(
    pl.kernel,
    out_type=jax.ShapeDtypeStruct((NC, NPAD), jnp.float32),
    mesh=_mesh,
    scratch_types=[
        pltpu.VMEM((NPAD,), jnp.float32),
        pltpu.VMEM((CHUNK,), jnp.int32),
        pltpu.VMEM((CHUNK,), jnp.int32),
        pltpu.VMEM((CROWS, ROW), jnp.int32),
        pltpu.VMEM((CROWS, ROW), jnp.int32),
        pltpu.VMEM((CHUNK,), jnp.float32),
        pltpu.VMEM((CHUNK,), jnp.float32),
        pltpu.VMEM((SLICE,), jnp.float32),
        pltpu.VMEM_SHARED((NPAD,), jnp.float32),
        pltpu.SemaphoreType.DMA,
        pltpu.SemaphoreType.DMA,
        pltpu.SemaphoreType.DMA,
    ],
    compiler_params=pltpu.CompilerParams(needs_layout_passes=False),
)
def _edge_sum_kernel(src_hbm, dst_hbm, g_hbm, out_hbm, g_v, src0_v, src1_v,
                     dst0_v, dst1_v, val0_v, val1_v, zb_v, acc_sh, semA, semB,
                     sem_s):
    c = lax.axis_index("c")
    s = lax.axis_index("s")
    wid = c * NS + s

    pltpu.sync_copy(g_hbm, g_v)
    _init_acc_slice(zb_v, acc_sh, s)
    plsc.subcore_barrier()

    def gather_chunk(src_v, val_v):
        def gather_body(l, _):
            for u in range(4):
                o = (l * 4 + u) * LANES
                idx = src_v[pl.ds(o, LANES)]
                val_v[pl.ds(o, LANES)] = plsc.load_gather(g_v, [idx])
            return 0

        lax.fori_loop(0, CHUNK // (4 * LANES), gather_body, 0)

    def fire_scatters(dst_v, val_v):
        return [
            pltpu.async_copy(val_v.at[pl.ds(j * ROW, ROW)],
                             acc_sh.at[dst_v.at[j]], sem_s, add=True)
            for j in range(CROWS)
        ]

    def fire_fetch(k, src_v, dst_v, sem):
        for j in range(CROWS):
            pltpu.async_copy(src_hbm.at[k, j], src_v.at[pl.ds(j * ROW, ROW)],
                             sem)
        pltpu.async_copy(dst_hbm.at[k], dst_v, sem)

    def drain_fetch(k, src_v, dst_v, sem):
        for j in range(CROWS):
            pltpu.make_async_copy(src_hbm.at[k, j],
                                  src_v.at[pl.ds(j * ROW, ROW)], sem).wait()
        pltpu.make_async_copy(dst_hbm.at[k], dst_v, sem).wait()

    fire_fetch(wid, src0_v, dst0_v, semA)

    def pair_body(i, _):
        kA = wid + (2 * i) * NW
        kB = kA + NW
        kA2 = kA + 2 * NW

        @pl.when(kB < NCHT)
        def _():
            fire_fetch(kB, src1_v, dst1_v, semB)

        drain_fetch(kA, src0_v, dst0_v, semA)
        gather_chunk(src0_v, val0_v)
        descsA = fire_scatters(dst0_v, val0_v)

        @pl.when(kB < NCHT)
        def _():
            drain_fetch(kB, src1_v, dst1_v, semB)
            gather_chunk(src1_v, val1_v)
            descsB = fire_scatters(dst1_v, val1_v)
            for d in descsA:
                d.wait()

            @pl.when(kA2 < NCHT)
            def _():
                fire_fetch(kA2, src0_v, dst0_v, semA)

            for d in descsB:
                d.wait()

        @pl.when(kB >= NCHT)
        def _():
            for d in descsA:
                d.wait()

        return 0

    lax.fori_loop(0, (ITERS + 1) // 2, pair_body, 0)

    plsc.subcore_barrier()
    _writeout(zb_v, acc_sh, out_hbm, c, s)


def _node_prep_body(deg_parts_ref, x_ref, g_ref, dinv_ref):
    deg = deg_parts_ref[0] + deg_parts_ref[1] + 1.0
    dinv = lax.rsqrt(deg)
    dinv_ref[...] = dinv
    g_ref[...] = x_ref[...] * dinv


_node_prep = pl.pallas_call(
    _node_prep_body,
    out_shape=(
        jax.ShapeDtypeStruct((ROWS2D, 128), jnp.float32),
        jax.ShapeDtypeStruct((ROWS2D, 128), jnp.float32),
    ),
)


def _node_final_body(t_parts_ref, g_ref, dinv_ref, x_ref, w1_ref, b1_ref,
                     w2_ref, b2_ref, out_ref):
    t = t_parts_ref[0] + t_parts_ref[1]
    sc = dinv_ref[...] * (t + g_ref[...])
    acc = jnp.zeros((ROWS2D, 128), jnp.float32)
    for j in range(H):
        h = jnp.maximum(sc * w1_ref[0, j] + b1_ref[0, j], 0.0)
        acc = acc + h * w2_ref[0, j]
    out_ref[...] = x_ref[...] + acc + b2_ref[0, 0]


_node_final = pl.pallas_call(
    _node_final_body,
    in_specs=[
        pl.BlockSpec(memory_space=pltpu.VMEM),
        pl.BlockSpec(memory_space=pltpu.VMEM),
        pl.BlockSpec(memory_space=pltpu.VMEM),
        pl.BlockSpec(memory_space=pltpu.VMEM),
        pl.BlockSpec(memory_space=pltpu.SMEM),
        pl.BlockSpec(memory_space=pltpu.SMEM),
        pl.BlockSpec(memory_space=pltpu.SMEM),
        pl.BlockSpec(memory_space=pltpu.SMEM),
    ],
    out_shape=jax.ShapeDtypeStruct((ROWS2D, 128), jnp.float32),
)


def kernel(x, edge_index, W1, b1, W2, b2):
    srcr = edge_index[0].astype(jnp.int32).reshape(NCHT, CROWS, ROW)
    dst = edge_index[1].astype(jnp.int32).reshape(NCHT, CROWS, ROW)
    x_flat = x[:, 0]
    x_pad = jnp.zeros((NPAD,), jnp.float32).at[:N].set(x_flat)

    deg_parts = _deg_kernel(dst)
    g, dinv = _node_prep(deg_parts.reshape(NC, ROWS2D, 128),
                         x_pad.reshape(ROWS2D, 128))

    t_parts = _edge_sum_kernel(srcr, dst, g.reshape(NPAD))

    out_pad = _node_final(
        t_parts.reshape(NC, ROWS2D, 128),
        g,
        dinv,
        x_pad.reshape(ROWS2D, 128),
        W1.reshape(1, H),
        b1.reshape(1, H),
        W2.reshape(1, H),
        b2.reshape(1, 1),
    )
    return out_pad.reshape(NPAD)[:N].reshape(N, 1)


# speedup vs baseline: 519.6038x; 1.0562x over previous
import functools

import jax
import jax.numpy as jnp
from jax import lax
from jax.experimental import pallas as pl
from jax.experimental.pallas import tpu as pltpu
from jax.experimental.pallas import tpu_sc as plsc

N = 100000
E = 6400000
H = 32

NPAD = 102400
NC = 2
NS = 16
NW = NC * NS
ROW = 128
CROWS = 16
CHUNK = CROWS * ROW
NCHT = E // CHUNK
ITERS = -(-NCHT // NW)
SLICE = NPAD // NS
LANES = 16

ROWS2D = NPAD // 128

_mesh = plsc.VectorSubcoreMesh(core_axis_name="c", subcore_axis_name="s")


def _fill1d(ref, n, value):
    v = jnp.full((LANES,), value, jnp.float32)

    def body(i, _):
        ref[pl.ds(i * LANES, LANES)] = v
        return 0

    lax.fori_loop(0, n // LANES, body, 0)


def _init_acc_slice(zb_v, acc_sh, s):
    _fill1d(zb_v, SLICE, 0.0)
    pltpu.sync_copy(zb_v, acc_sh.at[pl.ds(s * SLICE, SLICE)])


def _writeout(zb_v, acc_sh, out_hbm, c, s):
    pltpu.sync_copy(acc_sh.at[pl.ds(s * SLICE, SLICE)], zb_v)
    pltpu.sync_copy(zb_v, out_hbm.at[c, pl.ds(s * SLICE, SLICE)])


@functools.partial(
    pl.kernel,
    out_type=jax.ShapeDtypeStruct((NC, NPAD), jnp.float32),
    mesh=_mesh,
    scratch_types=[
        pltpu.VMEM((CROWS, ROW), jnp.int32),
        pltpu.VMEM((CROWS, ROW), jnp.int32),
        pltpu.VMEM((CHUNK,), jnp.float32),
        pltpu.VMEM((SLICE,), jnp.float32),
        pltpu.VMEM_SHARED((NPAD,), jnp.float32),
        pltpu.SemaphoreType.DMA,
        pltpu.SemaphoreType.DMA,
        pltpu.SemaphoreType.DMA,
    ],
    compiler_params=pltpu.CompilerParams(needs_layout_passes=False),
)
def _deg_kernel(ei_hbm, out_hbm, dst0_v, dst1_v, ones_v, zb_v, acc_sh, semA,
                semB, sem_s):
    dst_hbm = ei_hbm.at[1]
    c = lax.axis_index("c")
    s = lax.axis_index("s")
    wid = c * NS + s

    _init_acc_slice(zb_v, acc_sh, s)
    _fill1d(ones_v, CHUNK, 1.0)
    plsc.subcore_barrier()

    def fire_scatters(dst_v):
        return [
            pltpu.async_copy(ones_v.at[pl.ds(j * ROW, ROW)],
                             acc_sh.at[dst_v.at[j]], sem_s, add=True)
            for j in range(CROWS)
        ]

    pltpu.async_copy(dst_hbm.at[wid], dst0_v, semA)

    def pair_body(i, _):
        kA = wid + (2 * i) * NW
        kB = kA + NW
        kA2 = kA + 2 * NW

        @pl.when(kB < NCHT)
        def _():
            pltpu.async_copy(dst_hbm.at[kB], dst1_v, semB)

        pltpu.make_async_copy(dst_hbm.at[kA], dst0_v, semA).wait()
        descsA = fire_scatters(dst0_v)

        @pl.when(kB < NCHT)
        def _():
            pltpu.make_async_copy(dst_hbm.at[kB], dst1_v, semB).wait()
            descsB = fire_scatters(dst1_v)
            for d in descsA:
                d.wait()

            @pl.when(kA2 < NCHT)
            def _():
                pltpu.async_copy(dst_hbm.at[kA2], dst0_v, semA)

            for d in descsB:
                d.wait()

        @pl.when(kB >= NCHT)
        def _():
            for d in descsA:
                d.wait()

        return 0

    lax.fori_loop(0, (ITERS + 1) // 2, pair_body, 0)

    plsc.subcore_barrier()
    _writeout(zb_v, acc_sh, out_hbm, c, s)


@functools.partial(
    pl.kernel,
    out_type=jax.ShapeDtypeStruct((NC, NPAD), jnp.float32),
    mesh=_mesh,
    scratch_types=[
        pltpu.VMEM((NPAD,), jnp.float32),
        pltpu.VMEM((CHUNK,), jnp.int32),
        pltpu.VMEM((CHUNK,), jnp.int32),
        pltpu.VMEM((CROWS, ROW), jnp.int32),
        pltpu.VMEM((CROWS, ROW), jnp.int32),
        pltpu.VMEM((CHUNK,), jnp.float32),
        pltpu.VMEM((CHUNK,), jnp.float32),
        pltpu.VMEM((SLICE,), jnp.float32),
        pltpu.VMEM_SHARED((NPAD,), jnp.float32),
        pltpu.SemaphoreType.DMA,
        pltpu.SemaphoreType.DMA,
        pltpu.SemaphoreType.DMA,
    ],
    compiler_params=pltpu.CompilerParams(needs_layout_passes=False),
)
def _edge_sum_kernel(ei_hbm, g_hbm, out_hbm, g_v, src0_v, src1_v,
                     dst0_v, dst1_v, val0_v, val1_v, zb_v, acc_sh, semA, semB,
                     sem_s):
    src_hbm = ei_hbm.at[0]
    dst_hbm = ei_hbm.at[1]
    c = lax.axis_index("c")
    s = lax.axis_index("s")
    wid = c * NS + s

    pltpu.sync_copy(g_hbm, g_v)
    _init_acc_slice(zb_v, acc_sh, s)
    plsc.subcore_barrier()

    def gather_chunk(src_v, val_v):
        def gather_body(l, _):
            for u in range(4):
                o = (l * 4 + u) * LANES
                idx = src_v[pl.ds(o, LANES)]
                val_v[pl.ds(o, LANES)] = plsc.load_gather(g_v, [idx])
            return 0

        lax.fori_loop(0, CHUNK // (4 * LANES), gather_body, 0)

    def fire_scatters(dst_v, val_v):
        return [
            pltpu.async_copy(val_v.at[pl.ds(j * ROW, ROW)],
                             acc_sh.at[dst_v.at[j]], sem_s, add=True)
            for j in range(CROWS)
        ]

    def fire_fetch(k, src_v, dst_v, sem):
        for j in range(CROWS):
            pltpu.async_copy(src_hbm.at[k, j], src_v.at[pl.ds(j * ROW, ROW)],
                             sem)
        pltpu.async_copy(dst_hbm.at[k], dst_v, sem)

    def drain_fetch(k, src_v, dst_v, sem):
        for j in range(CROWS):
            pltpu.make_async_copy(src_hbm.at[k, j],
                                  src_v.at[pl.ds(j * ROW, ROW)], sem).wait()
        pltpu.make_async_copy(dst_hbm.at[k], dst_v, sem).wait()

    fire_fetch(wid, src0_v, dst0_v, semA)

    def pair_body(i, _):
        kA = wid + (2 * i) * NW
        kB = kA + NW
        kA2 = kA + 2 * NW

        @pl.when(kB < NCHT)
        def _():
            fire_fetch(kB, src1_v, dst1_v, semB)

        drain_fetch(kA, src0_v, dst0_v, semA)
        gather_chunk(src0_v, val0_v)
        descsA = fire_scatters(dst0_v, val0_v)

        @pl.when(kB < NCHT)
        def _():
            drain_fetch(kB, src1_v, dst1_v, semB)
            gather_chunk(src1_v, val1_v)
            descsB = fire_scatters(dst1_v, val1_v)
            for d in descsA:
                d.wait()

            @pl.when(kA2 < NCHT)
            def _():
                fire_fetch(kA2, src0_v, dst0_v, semA)

            for d in descsB:
                d.wait()

        @pl.when(kB >= NCHT)
        def _():
            for d in descsA:
                d.wait()

        return 0

    lax.fori_loop(0, (ITERS + 1) // 2, pair_body, 0)

    plsc.subcore_barrier()
    _writeout(zb_v, acc_sh, out_hbm, c, s)


def _node_prep_body(deg_parts_ref, x_ref, g_ref, dinv_ref):
    deg = deg_parts_ref[0] + deg_parts_ref[1] + 1.0
    dinv = lax.rsqrt(deg)
    dinv_ref[...] = dinv
    g_ref[...] = x_ref[...] * dinv


_node_prep = pl.pallas_call(
    _node_prep_body,
    out_shape=(
        jax.ShapeDtypeStruct((ROWS2D, 128), jnp.float32),
        jax.ShapeDtypeStruct((ROWS2D, 128), jnp.float32),
    ),
)


def _node_final_body(t_parts_ref, g_ref, dinv_ref, x_ref, w1_ref, b1_ref,
                     w2_ref, b2_ref, out_ref):
    t = t_parts_ref[0] + t_parts_ref[1]
    sc = dinv_ref[...] * (t + g_ref[...])
    acc = jnp.zeros((ROWS2D, 128), jnp.float32)
    for j in range(H):
        h = jnp.maximum(sc * w1_ref[0, j] + b1_ref[0, j], 0.0)
        acc = acc + h * w2_ref[0, j]
    out_ref[...] = x_ref[...] + acc + b2_ref[0, 0]


_node_final = pl.pallas_call(
    _node_final_body,
    in_specs=[
        pl.BlockSpec(memory_space=pltpu.VMEM),
        pl.BlockSpec(memory_space=pltpu.VMEM),
        pl.BlockSpec(memory_space=pltpu.VMEM),
        pl.BlockSpec(memory_space=pltpu.VMEM),
        pl.BlockSpec(memory_space=pltpu.SMEM),
        pl.BlockSpec(memory_space=pltpu.SMEM),
        pl.BlockSpec(memory_space=pltpu.SMEM),
        pl.BlockSpec(memory_space=pltpu.SMEM),
    ],
    out_shape=jax.ShapeDtypeStruct((ROWS2D, 128), jnp.float32),
)


def kernel(x, edge_index, W1, b1, W2, b2):
    ei4 = edge_index.astype(jnp.int32).reshape(2, NCHT, CROWS, ROW)
    x_flat = x[:, 0]
    x_pad = jnp.zeros((NPAD,), jnp.float32).at[:N].set(x_flat)

    deg_parts = _deg_kernel(ei4)
    g, dinv = _node_prep(deg_parts.reshape(NC, ROWS2D, 128),
                         x_pad.reshape(ROWS2D, 128))

    t_parts = _edge_sum_kernel(ei4, g.reshape(NPAD))

    out_pad = _node_final(
        t_parts.reshape(NC, ROWS2D, 128),
        g,
        dinv,
        x_pad.reshape(ROWS2D, 128),
        W1.reshape(1, H),
        b1.reshape(1, H),
        W2.reshape(1, H),
        b2.reshape(1, 1),
    )
    return out_pad.reshape(NPAD)[:N].reshape(N, 1)


# speedup vs baseline: 549.5460x; 1.0576x over previous
import functools

import jax
import jax.numpy as jnp
from jax import lax
from jax.experimental import pallas as pl
from jax.experimental.pallas import tpu as pltpu
from jax.experimental.pallas import tpu_sc as plsc

N = 100000
E = 6400000
H = 32

NPAD = 102400
NC = 2
NS = 16
NW = NC * NS
ROW = 128
CROWS = 16
CHUNK = CROWS * ROW
NCHT = E // CHUNK
ITERS = -(-NCHT // NW)
SLICE = NPAD // NS
LANES = 16

ROWS2D = NPAD // 128

_mesh = plsc.VectorSubcoreMesh(core_axis_name="c", subcore_axis_name="s")


def _fill1d(ref, n, value):
    v = jnp.full((LANES,), value, jnp.float32)

    def body(i, _):
        ref[pl.ds(i * LANES, LANES)] = v
        return 0

    lax.fori_loop(0, n // LANES, body, 0)


def _init_acc_slice(zb_v, acc_sh, s):
    _fill1d(zb_v, SLICE, 0.0)
    pltpu.sync_copy(zb_v, acc_sh.at[pl.ds(s * SLICE, SLICE)])


def _writeout(zb_v, acc_sh, out_hbm, c, s):
    pltpu.sync_copy(acc_sh.at[pl.ds(s * SLICE, SLICE)], zb_v)
    pltpu.sync_copy(zb_v, out_hbm.at[c, pl.ds(s * SLICE, SLICE)])


@functools.partial(
    pl.kernel,
    out_type=jax.ShapeDtypeStruct((NC, NPAD), jnp.float32),
    mesh=_mesh,
    scratch_types=[
        pltpu.VMEM((CROWS, ROW), jnp.int32),
        pltpu.VMEM((CROWS, ROW), jnp.int32),
        pltpu.VMEM((CHUNK,), jnp.float32),
        pltpu.VMEM((SLICE,), jnp.float32),
        pltpu.VMEM_SHARED((NPAD,), jnp.float32),
        pltpu.SemaphoreType.DMA,
        pltpu.SemaphoreType.DMA,
        pltpu.SemaphoreType.DMA,
    ],
    compiler_params=pltpu.CompilerParams(needs_layout_passes=False),
)
def _deg_kernel(ei_hbm, out_hbm, dst0_v, dst1_v, ones_v, zb_v, acc_sh, semA,
                semB, sem_s):
    dst_hbm = ei_hbm.at[1]
    c = lax.axis_index("c")
    s = lax.axis_index("s")
    wid = c * NS + s

    _init_acc_slice(zb_v, acc_sh, s)
    _fill1d(ones_v, CHUNK, 1.0)
    plsc.subcore_barrier()

    def fire_scatters(dst_v):
        return [
            pltpu.async_copy(ones_v.at[pl.ds(j * ROW, ROW)],
                             acc_sh.at[dst_v.at[j]], sem_s, add=True)
            for j in range(CROWS)
        ]

    pltpu.async_copy(dst_hbm.at[wid], dst0_v, semA)

    def pair_body(i, _):
        kA = wid + (2 * i) * NW
        kB = kA + NW
        kA2 = kA + 2 * NW

        @pl.when(kB < NCHT)
        def _():
            pltpu.async_copy(dst_hbm.at[kB], dst1_v, semB)

        pltpu.make_async_copy(dst_hbm.at[kA], dst0_v, semA).wait()
        descsA = fire_scatters(dst0_v)

        @pl.when(kB < NCHT)
        def _():
            pltpu.make_async_copy(dst_hbm.at[kB], dst1_v, semB).wait()
            descsB = fire_scatters(dst1_v)
            for d in descsA:
                d.wait()

            @pl.when(kA2 < NCHT)
            def _():
                pltpu.async_copy(dst_hbm.at[kA2], dst0_v, semA)

            for d in descsB:
                d.wait()

        @pl.when(kB >= NCHT)
        def _():
            for d in descsA:
                d.wait()

        return 0

    lax.fori_loop(0, (ITERS + 1) // 2, pair_body, 0)

    plsc.subcore_barrier()
    _writeout(zb_v, acc_sh, out_hbm, c, s)


@functools.partial(
    pl.kernel,
    out_type=jax.ShapeDtypeStruct((NC, NPAD), jnp.float32),
    mesh=_mesh,
    scratch_types=[
        pltpu.VMEM((NPAD,), jnp.float32),
        pltpu.VMEM((CHUNK,), jnp.int32),
        pltpu.VMEM((CHUNK,), jnp.int32),
        pltpu.VMEM((CHUNK,), jnp.int32),
        pltpu.VMEM((CROWS, ROW), jnp.int32),
        pltpu.VMEM((CROWS, ROW), jnp.int32),
        pltpu.VMEM((CROWS, ROW), jnp.int32),
        pltpu.VMEM((CHUNK,), jnp.float32),
        pltpu.VMEM((CHUNK,), jnp.float32),
        pltpu.VMEM((CHUNK,), jnp.float32),
        pltpu.VMEM_SHARED((NPAD,), jnp.float32),
        pltpu.SemaphoreType.DMA,
        pltpu.SemaphoreType.DMA,
        pltpu.SemaphoreType.DMA,
        pltpu.SemaphoreType.DMA,
    ],
    compiler_params=pltpu.CompilerParams(needs_layout_passes=False),
)
def _edge_sum_kernel(ei_hbm, g_hbm, out_hbm, g_v, src0_v, src1_v, src2_v,
                     dst0_v, dst1_v, dst2_v, val0_v, val1_v, val2_v, acc_sh,
                     semA, semB, semC, sem_s):
    src_hbm = ei_hbm.at[0]
    dst_hbm = ei_hbm.at[1]
    c = lax.axis_index("c")
    s = lax.axis_index("s")
    wid = c * NS + s

    pltpu.sync_copy(g_hbm, g_v)

    _fill1d(val0_v, CHUNK, 0.0)
    for q in range(3):
        pltpu.sync_copy(val0_v,
                        acc_sh.at[pl.ds(s * SLICE + q * CHUNK, CHUNK)])
    pltpu.sync_copy(val0_v.at[pl.ds(0, SLICE - 3 * CHUNK)],
                    acc_sh.at[pl.ds(s * SLICE + 3 * CHUNK, SLICE - 3 * CHUNK)])
    plsc.subcore_barrier()

    def gather_chunk(src_v, val_v):
        def gather_body(l, _):
            for u in range(4):
                o = (l * 4 + u) * LANES
                idx = src_v[pl.ds(o, LANES)]
                val_v[pl.ds(o, LANES)] = plsc.load_gather(g_v, [idx])
            return 0

        lax.fori_loop(0, CHUNK // (4 * LANES), gather_body, 0)

    def fire_scatters(dst_v, val_v):
        return [
            pltpu.async_copy(val_v.at[pl.ds(j * ROW, ROW)],
                             acc_sh.at[dst_v.at[j]], sem_s, add=True)
            for j in range(CROWS)
        ]

    def fire_fetch(k, src_v, dst_v, sem):
        for j in range(CROWS):
            pltpu.async_copy(src_hbm.at[k, j], src_v.at[pl.ds(j * ROW, ROW)],
                             sem)
        pltpu.async_copy(dst_hbm.at[k], dst_v, sem)

    def drain_fetch(k, src_v, dst_v, sem):
        for j in range(CROWS):
            pltpu.make_async_copy(src_hbm.at[k, j],
                                  src_v.at[pl.ds(j * ROW, ROW)], sem).wait()
        pltpu.make_async_copy(dst_hbm.at[k], dst_v, sem).wait()

    def process(k, src_v, dst_v, val_v, sem):
        drain_fetch(k, src_v, dst_v, sem)
        gather_chunk(src_v, val_v)
        return fire_scatters(dst_v, val_v)

    fire_fetch(wid, src0_v, dst0_v, semA)
    fire_fetch(wid + NW, src1_v, dst1_v, semB)

    def triple_body(i, _):
        kA = wid + (3 * i) * NW
        kB = kA + NW
        kC = kA + 2 * NW
        kD = kA + 3 * NW
        kE = kA + 4 * NW

        descsA = process(kA, src0_v, dst0_v, val0_v, semA)

        @pl.when(kC < NCHT)
        def _():
            fire_fetch(kC, src2_v, dst2_v, semC)

        @pl.when(kB < NCHT)
        def _():
            descsB = process(kB, src1_v, dst1_v, val1_v, semB)
            for d in descsA:
                d.wait()

            @pl.when(kD < NCHT)
            def _():
                fire_fetch(kD, src0_v, dst0_v, semA)

            @pl.when(kC < NCHT)
            def _():
                descsC = process(kC, src2_v, dst2_v, val2_v, semC)
                for d in descsB:
                    d.wait()

                @pl.when(kE < NCHT)
                def _():
                    fire_fetch(kE, src1_v, dst1_v, semB)

                for d in descsC:
                    d.wait()

            @pl.when(kC >= NCHT)
            def _():
                for d in descsB:
                    d.wait()

        @pl.when(kB >= NCHT)
        def _():
            for d in descsA:
                d.wait()

        return 0

    lax.fori_loop(0, (ITERS + 2) // 3, triple_body, 0)

    plsc.subcore_barrier()

    for q in range(3):
        pltpu.sync_copy(acc_sh.at[pl.ds(s * SLICE + q * CHUNK, CHUNK)], val0_v)
        pltpu.sync_copy(val0_v, out_hbm.at[c, pl.ds(s * SLICE + q * CHUNK,
                                                    CHUNK)])
    pltpu.sync_copy(acc_sh.at[pl.ds(s * SLICE + 3 * CHUNK, SLICE - 3 * CHUNK)],
                    val0_v.at[pl.ds(0, SLICE - 3 * CHUNK)])
    pltpu.sync_copy(val0_v.at[pl.ds(0, SLICE - 3 * CHUNK)],
                    out_hbm.at[c, pl.ds(s * SLICE + 3 * CHUNK,
                                        SLICE - 3 * CHUNK)])


def _node_prep_body(deg_parts_ref, x_ref, g_ref, dinv_ref):
    deg = deg_parts_ref[0] + deg_parts_ref[1] + 1.0
    dinv = lax.rsqrt(deg)
    dinv_ref[...] = dinv
    g_ref[...] = x_ref[...] * dinv


_node_prep = pl.pallas_call(
    _node_prep_body,
    out_shape=(
        jax.ShapeDtypeStruct((ROWS2D, 128), jnp.float32),
        jax.ShapeDtypeStruct((ROWS2D, 128), jnp.float32),
    ),
)


def _node_final_body(t_parts_ref, g_ref, dinv_ref, x_ref, w1_ref, b1_ref,
                     w2_ref, b2_ref, out_ref):
    t = t_parts_ref[0] + t_parts_ref[1]
    sc = dinv_ref[...] * (t + g_ref[...])
    acc = jnp.zeros((ROWS2D, 128), jnp.float32)
    for j in range(H):
        h = jnp.maximum(sc * w1_ref[0, j] + b1_ref[0, j], 0.0)
        acc = acc + h * w2_ref[0, j]
    out_ref[...] = x_ref[...] + acc + b2_ref[0, 0]


_node_final = pl.pallas_call(
    _node_final_body,
    in_specs=[
        pl.BlockSpec(memory_space=pltpu.VMEM),
        pl.BlockSpec(memory_space=pltpu.VMEM),
        pl.BlockSpec(memory_space=pltpu.VMEM),
        pl.BlockSpec(memory_space=pltpu.VMEM),
        pl.BlockSpec(memory_space=pltpu.SMEM),
        pl.BlockSpec(memory_space=pltpu.SMEM),
        pl.BlockSpec(memory_space=pltpu.SMEM),
        pl.BlockSpec(memory_space=pltpu.SMEM),
    ],
    out_shape=jax.ShapeDtypeStruct((ROWS2D, 128), jnp.float32),
)


def kernel(x, edge_index, W1, b1, W2, b2):
    ei4 = edge_index.astype(jnp.int32).reshape(2, NCHT, CROWS, ROW)
    x_flat = x[:, 0]
    x_pad = jnp.zeros((NPAD,), jnp.float32).at[:N].set(x_flat)

    deg_parts = _deg_kernel(ei4)
    g, dinv = _node_prep(deg_parts.reshape(NC, ROWS2D, 128),
                         x_pad.reshape(ROWS2D, 128))

    t_parts = _edge_sum_kernel(ei4, g.reshape(NPAD))

    out_pad = _node_final(
        t_parts.reshape(NC, ROWS2D, 128),
        g,
        dinv,
        x_pad.reshape(ROWS2D, 128),
        W1.reshape(1, H),
        b1.reshape(1, H),
        W2.reshape(1, H),
        b2.reshape(1, 1),
    )
    return out_pad.reshape(NPAD)[:N].reshape(N, 1)


# speedup vs baseline: 650.2144x; 1.1832x over previous
import functools

import jax
import jax.numpy as jnp
from jax import lax
from jax.experimental import pallas as pl
from jax.experimental.pallas import tpu as pltpu
from jax.experimental.pallas import tpu_sc as plsc

N = 100000
E = 6400000
H = 32

NPAD = 102400
NC = 2
NS = 16
NW = NC * NS
ROW = 128
CROWS = 16
CHUNK = CROWS * ROW
NCHT = E // CHUNK
ITERS = -(-NCHT // NW)
SLICE = NPAD // NS
LANES = 16

ROWS2D = NPAD // 128

_mesh = plsc.VectorSubcoreMesh(core_axis_name="c", subcore_axis_name="s")


def _fill1d(ref, n, value):
    v = jnp.full((LANES,), value, jnp.float32)

    def body(i, _):
        ref[pl.ds(i * LANES, LANES)] = v
        return 0

    lax.fori_loop(0, n // LANES, body, 0)


def _init_acc_slice(zb_v, acc_sh, s):
    _fill1d(zb_v, SLICE, 0.0)
    pltpu.sync_copy(zb_v, acc_sh.at[pl.ds(s * SLICE, SLICE)])


def _writeout(zb_v, acc_sh, out_hbm, c, s):
    pltpu.sync_copy(acc_sh.at[pl.ds(s * SLICE, SLICE)], zb_v)
    pltpu.sync_copy(zb_v, out_hbm.at[c, pl.ds(s * SLICE, SLICE)])


@functools.partial(
    pl.kernel,
    out_type=jax.ShapeDtypeStruct((NC, NPAD), jnp.float32),
    mesh=_mesh,
    scratch_types=[
        pltpu.VMEM((CROWS, ROW), jnp.int32),
        pltpu.VMEM((CROWS, ROW), jnp.int32),
        pltpu.VMEM((CHUNK,), jnp.float32),
        pltpu.VMEM((SLICE,), jnp.float32),
        pltpu.VMEM_SHARED((NPAD,), jnp.float32),
        pltpu.SemaphoreType.DMA,
        pltpu.SemaphoreType.DMA,
        pltpu.SemaphoreType.DMA,
    ],
    compiler_params=pltpu.CompilerParams(needs_layout_passes=False),
)
def _deg_kernel(ei_hbm, out_hbm, dst0_v, dst1_v, ones_v, zb_v, acc_sh, semA,
                semB, sem_s):
    c = lax.axis_index("c")
    s = lax.axis_index("s")
    wid = c * NS + s

    _init_acc_slice(zb_v, acc_sh, s)
    _fill1d(ones_v, CHUNK, 1.0)
    plsc.subcore_barrier()

    def fire_scatters(dst_v):
        return [
            pltpu.async_copy(ones_v.at[pl.ds(j * ROW, ROW)],
                             acc_sh.at[dst_v.at[j]], sem_s, add=True)
            for j in range(CROWS)
        ]

    def fire_fetch(k, dst_v, sem):
        for j in range(CROWS):
            pltpu.async_copy(ei_hbm.at[1, pl.ds(k * CHUNK + j * ROW, ROW)],
                             dst_v.at[j], sem)

    def drain_fetch(k, dst_v, sem):
        for j in range(CROWS):
            pltpu.make_async_copy(
                ei_hbm.at[1, pl.ds(k * CHUNK + j * ROW, ROW)], dst_v.at[j],
                sem).wait()

    fire_fetch(wid, dst0_v, semA)

    def pair_body(i, _):
        kA = wid + (2 * i) * NW
        kB = kA + NW
        kA2 = kA + 2 * NW

        @pl.when(kB < NCHT)
        def _():
            fire_fetch(kB, dst1_v, semB)

        drain_fetch(kA, dst0_v, semA)
        descsA = fire_scatters(dst0_v)

        @pl.when(kB < NCHT)
        def _():
            drain_fetch(kB, dst1_v, semB)
            descsB = fire_scatters(dst1_v)
            for d in descsA:
                d.wait()

            @pl.when(kA2 < NCHT)
            def _():
                fire_fetch(kA2, dst0_v, semA)

            for d in descsB:
                d.wait()

        @pl.when(kB >= NCHT)
        def _():
            for d in descsA:
                d.wait()

        return 0

    lax.fori_loop(0, (ITERS + 1) // 2, pair_body, 0)

    plsc.subcore_barrier()
    _writeout(zb_v, acc_sh, out_hbm, c, s)


@functools.partial(
    pl.kernel,
    out_type=jax.ShapeDtypeStruct((NC, NPAD), jnp.float32),
    mesh=_mesh,
    scratch_types=[
        pltpu.VMEM((NPAD,), jnp.float32),
        pltpu.VMEM((CHUNK,), jnp.int32),
        pltpu.VMEM((CHUNK,), jnp.int32),
        pltpu.VMEM((CHUNK,), jnp.int32),
        pltpu.VMEM((CROWS, ROW), jnp.int32),
        pltpu.VMEM((CROWS, ROW), jnp.int32),
        pltpu.VMEM((CROWS, ROW), jnp.int32),
        pltpu.VMEM((CHUNK,), jnp.float32),
        pltpu.VMEM((CHUNK,), jnp.float32),
        pltpu.VMEM((CHUNK,), jnp.float32),
        pltpu.VMEM_SHARED((NPAD,), jnp.float32),
        pltpu.SemaphoreType.DMA,
        pltpu.SemaphoreType.DMA,
        pltpu.SemaphoreType.DMA,
        pltpu.SemaphoreType.DMA,
    ],
    compiler_params=pltpu.CompilerParams(needs_layout_passes=False),
)
def _edge_sum_kernel(ei_hbm, g_hbm, out_hbm, g_v, src0_v, src1_v, src2_v,
                     dst0_v, dst1_v, dst2_v, val0_v, val1_v, val2_v, acc_sh,
                     semA, semB, semC, sem_s):
    c = lax.axis_index("c")
    s = lax.axis_index("s")
    wid = c * NS + s

    pltpu.sync_copy(g_hbm, g_v)

    _fill1d(val0_v, CHUNK, 0.0)
    for q in range(3):
        pltpu.sync_copy(val0_v,
                        acc_sh.at[pl.ds(s * SLICE + q * CHUNK, CHUNK)])
    pltpu.sync_copy(val0_v.at[pl.ds(0, SLICE - 3 * CHUNK)],
                    acc_sh.at[pl.ds(s * SLICE + 3 * CHUNK, SLICE - 3 * CHUNK)])
    plsc.subcore_barrier()

    def gather_chunk(src_v, val_v):
        def gather_body(l, _):
            for u in range(4):
                o = (l * 4 + u) * LANES
                idx = src_v[pl.ds(o, LANES)]
                val_v[pl.ds(o, LANES)] = plsc.load_gather(g_v, [idx])
            return 0

        lax.fori_loop(0, CHUNK // (4 * LANES), gather_body, 0)

    def fire_scatters(dst_v, val_v):
        return [
            pltpu.async_copy(val_v.at[pl.ds(j * ROW, ROW)],
                             acc_sh.at[dst_v.at[j]], sem_s, add=True)
            for j in range(CROWS)
        ]

    def fire_fetch(k, src_v, dst_v, sem):
        pltpu.async_copy(ei_hbm.at[0, pl.ds(k * CHUNK, CHUNK)], src_v, sem)
        for j in range(CROWS):
            pltpu.async_copy(ei_hbm.at[1, pl.ds(k * CHUNK + j * ROW, ROW)],
                             dst_v.at[j], sem)

    def drain_fetch(k, src_v, dst_v, sem):
        pltpu.make_async_copy(ei_hbm.at[0, pl.ds(k * CHUNK, CHUNK)], src_v,
                              sem).wait()
        for j in range(CROWS):
            pltpu.make_async_copy(
                ei_hbm.at[1, pl.ds(k * CHUNK + j * ROW, ROW)], dst_v.at[j],
                sem).wait()

    def process(k, src_v, dst_v, val_v, sem):
        drain_fetch(k, src_v, dst_v, sem)
        gather_chunk(src_v, val_v)
        return fire_scatters(dst_v, val_v)

    fire_fetch(wid, src0_v, dst0_v, semA)
    fire_fetch(wid + NW, src1_v, dst1_v, semB)

    def triple_body(i, _):
        kA = wid + (3 * i) * NW
        kB = kA + NW
        kC = kA + 2 * NW
        kD = kA + 3 * NW
        kE = kA + 4 * NW

        descsA = process(kA, src0_v, dst0_v, val0_v, semA)

        @pl.when(kC < NCHT)
        def _():
            fire_fetch(kC, src2_v, dst2_v, semC)

        @pl.when(kB < NCHT)
        def _():
            descsB = process(kB, src1_v, dst1_v, val1_v, semB)
            for d in descsA:
                d.wait()

            @pl.when(kD < NCHT)
            def _():
                fire_fetch(kD, src0_v, dst0_v, semA)

            @pl.when(kC < NCHT)
            def _():
                descsC = process(kC, src2_v, dst2_v, val2_v, semC)
                for d in descsB:
                    d.wait()

                @pl.when(kE < NCHT)
                def _():
                    fire_fetch(kE, src1_v, dst1_v, semB)

                for d in descsC:
                    d.wait()

            @pl.when(kC >= NCHT)
            def _():
                for d in descsB:
                    d.wait()

        @pl.when(kB >= NCHT)
        def _():
            for d in descsA:
                d.wait()

        return 0

    lax.fori_loop(0, (ITERS + 2) // 3, triple_body, 0)

    plsc.subcore_barrier()

    for q in range(3):
        pltpu.sync_copy(acc_sh.at[pl.ds(s * SLICE + q * CHUNK, CHUNK)], val0_v)
        pltpu.sync_copy(val0_v, out_hbm.at[c, pl.ds(s * SLICE + q * CHUNK,
                                                    CHUNK)])
    pltpu.sync_copy(acc_sh.at[pl.ds(s * SLICE + 3 * CHUNK, SLICE - 3 * CHUNK)],
                    val0_v.at[pl.ds(0, SLICE - 3 * CHUNK)])
    pltpu.sync_copy(val0_v.at[pl.ds(0, SLICE - 3 * CHUNK)],
                    out_hbm.at[c, pl.ds(s * SLICE + 3 * CHUNK,
                                        SLICE - 3 * CHUNK)])


def _node_prep_body(deg_parts_ref, x_ref, g_ref, dinv_ref):
    deg = deg_parts_ref[0] + deg_parts_ref[1] + 1.0
    dinv = lax.rsqrt(deg)
    dinv_ref[...] = dinv
    g_ref[...] = x_ref[...] * dinv


_node_prep = pl.pallas_call(
    _node_prep_body,
    out_shape=(
        jax.ShapeDtypeStruct((ROWS2D, 128), jnp.float32),
        jax.ShapeDtypeStruct((ROWS2D, 128), jnp.float32),
    ),
)


def _node_final_body(t_parts_ref, g_ref, dinv_ref, x_ref, w1_ref, b1_ref,
                     w2_ref, b2_ref, out_ref):
    t = t_parts_ref[0] + t_parts_ref[1]
    sc = dinv_ref[...] * (t + g_ref[...])
    acc = jnp.zeros((ROWS2D, 128), jnp.float32)
    for j in range(H):
        h = jnp.maximum(sc * w1_ref[0, j] + b1_ref[0, j], 0.0)
        acc = acc + h * w2_ref[0, j]
    out_ref[...] = x_ref[...] + acc + b2_ref[0, 0]


_node_final = pl.pallas_call(
    _node_final_body,
    in_specs=[
        pl.BlockSpec(memory_space=pltpu.VMEM),
        pl.BlockSpec(memory_space=pltpu.VMEM),
        pl.BlockSpec(memory_space=pltpu.VMEM),
        pl.BlockSpec(memory_space=pltpu.VMEM),
        pl.BlockSpec(memory_space=pltpu.SMEM),
        pl.BlockSpec(memory_space=pltpu.SMEM),
        pl.BlockSpec(memory_space=pltpu.SMEM),
        pl.BlockSpec(memory_space=pltpu.SMEM),
    ],
    out_shape=jax.ShapeDtypeStruct((ROWS2D, 128), jnp.float32),
)


def kernel(x, edge_index, W1, b1, W2, b2):
    ei2 = edge_index.astype(jnp.int32)
    x_flat = x[:, 0]
    x_pad = jnp.zeros((NPAD,), jnp.float32).at[:N].set(x_flat)

    deg_parts = _deg_kernel(ei2)
    g, dinv = _node_prep(deg_parts.reshape(NC, ROWS2D, 128),
                         x_pad.reshape(ROWS2D, 128))

    t_parts = _edge_sum_kernel(ei2, g.reshape(NPAD))

    out_pad = _node_final(
        t_parts.reshape(NC, ROWS2D, 128),
        g,
        dinv,
        x_pad.reshape(ROWS2D, 128),
        W1.reshape(1, H),
        b1.reshape(1, H),
        W2.reshape(1, H),
        b2.reshape(1, 1),
    )
    return out_pad.reshape(NPAD)[:N].reshape(N, 1)
